# Initial kernel scaffold; baseline (speedup 1.0000x reference)
import jax
import jax.numpy as jnp
from jax.experimental import pallas as pl


def kernel(features, tables):
    raise NotImplementedError("write your pallas kernel here")


# speedup vs baseline: 1.0466x; 1.0466x over previous
import functools

import jax
import jax.numpy as jnp
from jax import lax
from jax.experimental import pallas as pl
from jax.experimental.pallas import tpu as pltpu
from jax.experimental.pallas import tpu_sc as plsc

_BATCH = 16384
_FIELDS = 26
_VOCAB = 100000
_EMBED = 32
_TOTAL = _BATCH * _FIELDS
_NC = 2
_NS = 16
_NW = _NC * _NS
_PER_W = _TOTAL // _NW
_G = 128
_NG = _PER_W // _G


def _make_gather():
    mesh = plsc.VectorSubcoreMesh(core_axis_name="c", subcore_axis_name="s")

    @functools.partial(
        pl.kernel,
        mesh=mesh,
        out_type=jax.ShapeDtypeStruct((_TOTAL, _EMBED), jnp.float32),
        compiler_params=pltpu.CompilerParams(use_tc_tiling_on_sc=False),
        scratch_types=[
            pltpu.VMEM((_NG, _G), jnp.int32),
            pltpu.VMEM((_G, _EMBED), jnp.float32),
            pltpu.SemaphoreType.DMA,
        ],
    )
    def gather_kernel(idx_hbm, table_hbm, out_hbm, idx_v, rows_v, sem):
        wid = lax.axis_index("s") * _NC + lax.axis_index("c")
        base = wid * _PER_W
        pltpu.sync_copy(idx_hbm.at[pl.ds(wid * _NG, _NG)], idx_v)

        def body(i, carry):
            pltpu.async_copy(table_hbm.at[idx_v.at[i]], rows_v, sem).wait()
            pltpu.sync_copy(rows_v, out_hbm.at[pl.ds(base + i * _G, _G)])
            return carry

        lax.fori_loop(0, _NG, body, 0)

    return gather_kernel


_gather = _make_gather()


def kernel(features, tables):
    offs = jnp.arange(_FIELDS, dtype=jnp.int32) * _VOCAB
    flat_idx = (features.astype(jnp.int32) + offs[None, :]).reshape(
        _TOTAL // _G, _G)
    table_flat = tables.reshape(_FIELDS * _VOCAB, _EMBED)
    out = _gather(flat_idx, table_flat)
    return out.reshape(_BATCH, _FIELDS, _EMBED)


# speedup vs baseline: 1.0989x; 1.0499x over previous
import functools

import jax
import jax.numpy as jnp
from jax import lax
from jax.experimental import pallas as pl
from jax.experimental.pallas import tpu as pltpu
from jax.experimental.pallas import tpu_sc as plsc

_BATCH = 16384
_FIELDS = 26
_VOCAB = 100000
_EMBED = 32
_TOTAL = _BATCH * _FIELDS
_NC = 2
_NS = 16
_NW = _NC * _NS
_PER_W = _TOTAL // _NW
_G = 128
_NG = _PER_W // _G
_K = 13
_GROUP = _K * _G
_NGROUP = _NG // _K
_NPAIR = _NGROUP // 2


def _make_gather():
    mesh = plsc.VectorSubcoreMesh(core_axis_name="c", subcore_axis_name="s")

    @functools.partial(
        pl.kernel,
        mesh=mesh,
        out_type=jax.ShapeDtypeStruct((_TOTAL, _EMBED), jnp.float32),
        compiler_params=pltpu.CompilerParams(use_tc_tiling_on_sc=False),
        scratch_types=[
            pltpu.VMEM((_NG, _G), jnp.int32),
            pltpu.VMEM((2, _GROUP, _EMBED), jnp.float32),
            pltpu.SemaphoreType.DMA,
            pltpu.SemaphoreType.DMA,
            pltpu.SemaphoreType.DMA,
            pltpu.SemaphoreType.DMA,
        ],
    )
    def gather_kernel(idx_hbm, table_hbm, out_hbm, idx_v, rows_v,
                      gsem0, gsem1, wsem0, wsem1):
        wid = lax.axis_index("s") * _NC + lax.axis_index("c")
        base = wid * _PER_W
        pltpu.sync_copy(idx_hbm.at[pl.ds(wid * _NG, _NG)], idx_v)

        gsems = (gsem0, gsem1)
        wsems = (wsem0, wsem1)

        def body(gp, carry):
            for h in (0, 1):
                g = 2 * gp + h

                @pl.when(gp >= 1)
                def _():
                    pltpu.make_async_copy(
                        rows_v.at[h],
                        out_hbm.at[pl.ds(base, _GROUP)],
                        wsems[h]).wait()

                copies = []
                for b in range(_K):
                    copies.append(pltpu.async_copy(
                        table_hbm.at[idx_v.at[g * _K + b]],
                        rows_v.at[h, pl.ds(b * _G, _G)],
                        gsems[h]))
                for c in copies:
                    c.wait()

                pltpu.async_copy(
                    rows_v.at[h],
                    out_hbm.at[pl.ds(base + g * _GROUP, _GROUP)],
                    wsems[h])
            return carry

        lax.fori_loop(0, _NPAIR, body, 0)

        for h in (0, 1):
            pltpu.make_async_copy(
                rows_v.at[h],
                out_hbm.at[pl.ds(base, _GROUP)],
                wsems[h]).wait()

    return gather_kernel


_gather = _make_gather()


def kernel(features, tables):
    offs = jnp.arange(_FIELDS, dtype=jnp.int32) * _VOCAB
    flat_idx = (features.astype(jnp.int32) + offs[None, :]).reshape(
        _TOTAL // _G, _G)
    table_flat = tables.reshape(_FIELDS * _VOCAB, _EMBED)
    out = _gather(flat_idx, table_flat)
    return out.reshape(_BATCH, _FIELDS, _EMBED)


# speedup vs baseline: 2.0449x; 1.8609x over previous
import functools

import jax
import jax.numpy as jnp
from jax import lax
from jax.experimental import pallas as pl
from jax.experimental.pallas import tpu as pltpu
from jax.experimental.pallas import tpu_sc as plsc

_BATCH = 16384
_FIELDS = 26
_VOCAB = 100000
_EMBED = 32
_NC = 2
_NS = 16
_NW = _NC * _NS
_PLANES = _FIELDS * _EMBED
_PPW = _PLANES // _NW
_CHUNK = 4096
_NCHUNK = _BATCH // _CHUNK
_UNROLL = 4


def _make_plane_gather():
    mesh = plsc.VectorSubcoreMesh(core_axis_name="c", subcore_axis_name="s")

    @functools.partial(
        pl.kernel,
        mesh=mesh,
        out_type=jax.ShapeDtypeStruct((_FIELDS, _EMBED, _BATCH), jnp.float32),
        compiler_params=pltpu.CompilerParams(
            use_tc_tiling_on_sc=False, needs_layout_passes=False),
        scratch_types=[
            pltpu.VMEM((_VOCAB,), jnp.float32),
            pltpu.VMEM((_BATCH,), jnp.int32),
            pltpu.VMEM((2, _CHUNK), jnp.float32),
            pltpu.SemaphoreType.DMA,
            pltpu.SemaphoreType.DMA,
        ],
    )
    def plane_kernel(feat_hbm, tab_hbm, out_hbm, plane_v, idx_v, out_v,
                     wsem0, wsem1):
        wid = lax.axis_index("s") * _NC + lax.axis_index("c")
        wsems = (wsem0, wsem1)

        def plane_body(j, prev_f):
            p = wid * _PPW + j
            f = p // _EMBED
            e = p % _EMBED

            @pl.when(f != prev_f)
            def _():
                pltpu.sync_copy(feat_hbm.at[f], idx_v)

            pltpu.sync_copy(tab_hbm.at[f, e], plane_v)

            for c in range(_NCHUNK):
                b = c & 1

                if c >= 2:
                    pltpu.make_async_copy(
                        out_v.at[b],
                        out_hbm.at[f, e, pl.ds(0, _CHUNK)],
                        wsems[b]).wait()
                else:
                    @pl.when(j >= 1)
                    def _():
                        pltpu.make_async_copy(
                            out_v.at[b],
                            out_hbm.at[f, e, pl.ds(0, _CHUNK)],
                            wsems[b]).wait()

                def gather_body(i, carry):
                    base = c * _CHUNK + i * (16 * _UNROLL)
                    for u in range(_UNROLL):
                        vidx = idx_v[pl.ds(base + u * 16, 16)]
                        out_v[b, pl.ds(i * (16 * _UNROLL) + u * 16, 16)] = (
                            plsc.load_gather(plane_v, [vidx]))
                    return carry

                lax.fori_loop(0, _CHUNK // (16 * _UNROLL), gather_body, 0)

                pltpu.async_copy(
                    out_v.at[b],
                    out_hbm.at[f, e, pl.ds(c * _CHUNK, _CHUNK)],
                    wsems[b])
            return f

        lax.fori_loop(0, _PPW, plane_body, -1)

        for b in (0, 1):
            pltpu.make_async_copy(
                out_v.at[b],
                out_hbm.at[0, 0, pl.ds(0, _CHUNK)],
                wsems[b]).wait()

    return plane_kernel


_plane_gather = _make_plane_gather()


def kernel(features, tables):
    feat_t = features.astype(jnp.int32).T
    tab_t = jnp.transpose(tables, (0, 2, 1))
    out_t = _plane_gather(feat_t, tab_t)
    return jnp.transpose(out_t, (2, 0, 1))
